# baseline (speedup 1.0000x reference)
import functools

import jax
import jax.numpy as jnp
from jax import lax
from jax.experimental import pallas as pl
from jax.experimental.pallas import tpu as pltpu
from jax.experimental.pallas import tpu_sc as plsc

V = 100000
A = 1024
D = 64
B = 4096
S = 200
NTOK = B * S
NC = 2
NS = 16
NW = NC * NS
B_PER_W = B // NW

SEQ_PER_UNIT = 2
UNIT = SEQ_PER_UNIT * S
N_UNIT = B_PER_W // SEQ_PER_UNIT
SPLITS = ((0, 80), (80, 80), (160, 40))

FUS_PER_W = A // NW
MAIN_CHUNK = 400
MAIN_NCH = V // MAIN_CHUNK

_mesh = plsc.VectorSubcoreMesh(core_axis_name="c", subcore_axis_name="s")


def _wid():
    return lax.axis_index("s") * NC + lax.axis_index("c")


@functools.partial(
    pl.kernel,
    mesh=_mesh,
    out_type=jax.ShapeDtypeStruct((V + A, D), jnp.float32),
    scratch_types=[
        pltpu.VMEM((2, MAIN_CHUNK, D), jnp.float32),
        pltpu.SemaphoreType.DMA,
        pltpu.SemaphoreType.DMA,
        pltpu.SemaphoreType.DMA,
        pltpu.SemaphoreType.DMA,
    ],
)
def _build_concat(main_hbm, fus_hbm, out_hbm, buf, lsem0, lsem1, ssem0, ssem1):
    w = _wid()
    lsems = (lsem0, lsem1)
    ssems = (ssem0, ssem1)
    loads = [None, None]
    stores = [None, None]
    for t in range(7):
        b = t % 2
        r0 = pl.multiple_of((t * NW + w) * MAIN_CHUNK, 8)
        if stores[b] is not None:
            stores[b].wait()
        loads[b] = pltpu.async_copy(main_hbm.at[pl.ds(r0, MAIN_CHUNK)], buf.at[b], lsems[b])
        loads[b].wait()
        stores[b] = pltpu.async_copy(buf.at[b], out_hbm.at[pl.ds(r0, MAIN_CHUNK)], ssems[b])
    stores[0].wait()
    stores[1].wait()
    @pl.when(w < MAIN_NCH - 7 * NW)
    def _():
        r0 = pl.multiple_of((7 * NW + w) * MAIN_CHUNK, 8)
        pltpu.sync_copy(main_hbm.at[pl.ds(r0, MAIN_CHUNK)], buf.at[0])
        pltpu.sync_copy(buf.at[0], out_hbm.at[pl.ds(r0, MAIN_CHUNK)])
    f0 = pl.multiple_of(w * FUS_PER_W, 8)
    pltpu.sync_copy(fus_hbm.at[pl.ds(f0, FUS_PER_W)], buf.at[1].at[pl.ds(0, FUS_PER_W)])
    pltpu.sync_copy(
        buf.at[1].at[pl.ds(0, FUS_PER_W)],
        out_hbm.at[pl.ds(pl.multiple_of(V + w * FUS_PER_W, 8), FUS_PER_W)],
    )


@functools.partial(
    pl.kernel,
    mesh=_mesh,
    out_type=jax.ShapeDtypeStruct((B, S, D), jnp.float32),
    compiler_params=pltpu.CompilerParams(use_tc_tiling_on_sc=False),
    scratch_types=[
        pltpu.VMEM((B_PER_W, S), jnp.int32),
        pltpu.VMEM((2, SEQ_PER_UNIT, S, D), jnp.float32),
        pltpu.SemaphoreType.DMA,
        pltpu.SemaphoreType.DMA,
        pltpu.SemaphoreType.DMA,
    ],
)
def _gather(table_hbm, idx_hbm, out_hbm, idx_v, rows_v, gsem, osem0, osem1):
    w = _wid()
    b_base = w * B_PER_W
    pltpu.sync_copy(idx_hbm.at[pl.ds(pl.multiple_of(b_base, 8), B_PER_W)], idx_v)
    osems = (osem0, osem1)

    def pair(i, _):
        for b in range(2):
            u = 2 * i + b
            @pl.when(i >= 1)
            def _():
                pltpu.make_async_copy(
                    rows_v.at[b], out_hbm.at[pl.ds(0, SEQ_PER_UNIT)], osems[b]
                ).wait()

            copies = []
            for k in range(SEQ_PER_UNIT):
                for (s0, n) in SPLITS:
                    copies.append(
                        pltpu.async_copy(
                            table_hbm.at[idx_v.at[u * SEQ_PER_UNIT + k, pl.ds(s0, n)]],
                            rows_v.at[b, k, pl.ds(s0, n)],
                            gsem,
                        )
                    )
            for c in copies:
                c.wait()
            b0 = b_base + u * SEQ_PER_UNIT
            pltpu.async_copy(rows_v.at[b], out_hbm.at[pl.ds(b0, SEQ_PER_UNIT)], osems[b])
        return _

    lax.fori_loop(0, N_UNIT // 2, pair, None)
    for b in range(2):
        pltpu.make_async_copy(
            rows_v.at[b], out_hbm.at[pl.ds(0, SEQ_PER_UNIT)], osems[b]
        ).wait()


def kernel(input, embedding_weight, fusion_weight):
    table = _build_concat(embedding_weight, fusion_weight)
    return _gather(table, input.astype(jnp.int32))


# speedup vs baseline: 1.0073x; 1.0073x over previous
import functools

import jax
import jax.numpy as jnp
from jax import lax
from jax.experimental import pallas as pl
from jax.experimental.pallas import tpu as pltpu
from jax.experimental.pallas import tpu_sc as plsc

V = 100000
A = 1024
D = 64
B = 4096
S = 200
NTOK = B * S
NC = 2
NS = 16
NW = NC * NS
TOK_PER_W = NTOK // NW

IDX_W = 128
IDX_ROWS_W = TOK_PER_W // IDX_W
UNIT = 640
GPU_ = UNIT // IDX_W
N_UNIT = TOK_PER_W // UNIT

FUS_PER_W = A // NW
MAIN_CHUNK = 400
MAIN_NCH = V // MAIN_CHUNK

_mesh = plsc.VectorSubcoreMesh(core_axis_name="c", subcore_axis_name="s")


def _wid():
    return lax.axis_index("s") * NC + lax.axis_index("c")


@functools.partial(
    pl.kernel,
    mesh=_mesh,
    out_type=jax.ShapeDtypeStruct((V + A, D), jnp.float32),
    scratch_types=[
        pltpu.VMEM((2, MAIN_CHUNK, D), jnp.float32),
        pltpu.SemaphoreType.DMA,
        pltpu.SemaphoreType.DMA,
        pltpu.SemaphoreType.DMA,
        pltpu.SemaphoreType.DMA,
    ],
)
def _build_concat(main_hbm, fus_hbm, out_hbm, buf, lsem0, lsem1, ssem0, ssem1):
    w = _wid()
    lsems = (lsem0, lsem1)
    ssems = (ssem0, ssem1)
    loads = [None, None]
    stores = [None, None]
    for t in range(7):
        b = t % 2
        r0 = pl.multiple_of((t * NW + w) * MAIN_CHUNK, 8)
        if stores[b] is not None:
            stores[b].wait()
        loads[b] = pltpu.async_copy(main_hbm.at[pl.ds(r0, MAIN_CHUNK)], buf.at[b], lsems[b])
        loads[b].wait()
        stores[b] = pltpu.async_copy(buf.at[b], out_hbm.at[pl.ds(r0, MAIN_CHUNK)], ssems[b])
    stores[0].wait()
    stores[1].wait()
    @pl.when(w < MAIN_NCH - 7 * NW)
    def _():
        r0 = pl.multiple_of((7 * NW + w) * MAIN_CHUNK, 8)
        pltpu.sync_copy(main_hbm.at[pl.ds(r0, MAIN_CHUNK)], buf.at[0].at[pl.ds(0, MAIN_CHUNK)])
        pltpu.sync_copy(buf.at[0].at[pl.ds(0, MAIN_CHUNK)], out_hbm.at[pl.ds(r0, MAIN_CHUNK)])
    f0 = pl.multiple_of(w * FUS_PER_W, 8)
    pltpu.sync_copy(fus_hbm.at[pl.ds(f0, FUS_PER_W)], buf.at[1].at[pl.ds(0, FUS_PER_W)])
    pltpu.sync_copy(
        buf.at[1].at[pl.ds(0, FUS_PER_W)],
        out_hbm.at[pl.ds(pl.multiple_of(V + w * FUS_PER_W, 8), FUS_PER_W)],
    )


@functools.partial(
    pl.kernel,
    mesh=_mesh,
    out_type=jax.ShapeDtypeStruct((NTOK, D), jnp.float32),
    compiler_params=pltpu.CompilerParams(use_tc_tiling_on_sc=False),
    scratch_types=[
        pltpu.VMEM((IDX_ROWS_W, IDX_W), jnp.int32),
        pltpu.VMEM((2, UNIT, D), jnp.float32),
        pltpu.SemaphoreType.DMA,
        pltpu.SemaphoreType.DMA,
        pltpu.SemaphoreType.DMA,
    ],
)
def _gather(table_hbm, idx_hbm, out_hbm, idx_v, rows_v, gsem, osem0, osem1):
    w = _wid()
    base = w * TOK_PER_W
    pltpu.sync_copy(idx_hbm.at[pl.ds(pl.multiple_of(w * IDX_ROWS_W, 8), IDX_ROWS_W)], idx_v)
    osems = (osem0, osem1)

    def pair(i, _):
        for b in range(2):
            u = 2 * i + b
            @pl.when(i >= 1)
            def _():
                pltpu.make_async_copy(
                    rows_v.at[b], out_hbm.at[pl.ds(0, UNIT)], osems[b]
                ).wait()

            copies = [
                pltpu.async_copy(
                    table_hbm.at[idx_v.at[u * GPU_ + j]],
                    rows_v.at[b].at[pl.ds(j * IDX_W, IDX_W)],
                    gsem,
                )
                for j in range(GPU_)
            ]
            for c in copies:
                c.wait()
            t0 = pl.multiple_of(base + u * UNIT, 8)
            pltpu.async_copy(rows_v.at[b], out_hbm.at[pl.ds(t0, UNIT)], osems[b])
        return _

    lax.fori_loop(0, N_UNIT // 2, pair, None)
    for b in range(2):
        pltpu.make_async_copy(rows_v.at[b], out_hbm.at[pl.ds(0, UNIT)], osems[b]).wait()


def kernel(input, embedding_weight, fusion_weight):
    idx = input.reshape(NTOK // IDX_W, IDX_W).astype(jnp.int32)
    table = _build_concat(embedding_weight, fusion_weight)
    out = _gather(table, idx)
    return out.reshape(B, S, D)


# speedup vs baseline: 1.0231x; 1.0157x over previous
import functools

import jax
import jax.numpy as jnp
from jax import lax
from jax.experimental import pallas as pl
from jax.experimental.pallas import tpu as pltpu
from jax.experimental.pallas import tpu_sc as plsc

V = 100000
A = 1024
D = 64
B = 4096
S = 200
NTOK = B * S
NC = 2
NS = 16
NW = NC * NS
TOK_PER_W = NTOK // NW

IDX_W = 128
IDX_ROWS_W = TOK_PER_W // IDX_W
UNIT = 640
GPU_ = UNIT // IDX_W
N_UNIT = TOK_PER_W // UNIT

FUS_PER_S = A // NS
MAIN_CHUNK = 400
MAIN_NCH = V // MAIN_CHUNK

_mesh = plsc.VectorSubcoreMesh(core_axis_name="c", subcore_axis_name="s")


@functools.partial(
    pl.kernel,
    mesh=_mesh,
    out_type=(
        jax.ShapeDtypeStruct((NTOK, D), jnp.float32),
        jax.ShapeDtypeStruct((NC, V + A, D), jnp.float32),
    ),
    compiler_params=pltpu.CompilerParams(use_tc_tiling_on_sc=False),
    scratch_types=[
        pltpu.VMEM((2, UNIT, D), jnp.float32),
        pltpu.VMEM((IDX_ROWS_W, IDX_W), jnp.int32),
        pltpu.SemaphoreType.DMA,
        pltpu.SemaphoreType.DMA,
        pltpu.SemaphoreType.DMA,
        pltpu.SemaphoreType.DMA,
        pltpu.SemaphoreType.DMA,
    ],
)
def _fused(main_hbm, fus_hbm, idx_hbm, out_hbm, tab_hbm, buf, idx_v,
           lsem0, lsem1, ssem0, ssem1, gsem):
    c = lax.axis_index("c")
    s = lax.axis_index("s")
    w = s * NC + c
    lsems = (lsem0, lsem1)
    ssems = (ssem0, ssem1)
    tab = tab_hbm.at[c]

    loads = [None, None]
    stores = [None, None]
    for t in range(15):
        b = t % 2
        r0 = pl.multiple_of((t * NS + s) * MAIN_CHUNK, 8)
        if stores[b] is not None:
            stores[b].wait()
        loads[b] = pltpu.async_copy(
            main_hbm.at[pl.ds(r0, MAIN_CHUNK)], buf.at[b, pl.ds(0, MAIN_CHUNK)], lsems[b]
        )
        loads[b].wait()
        stores[b] = pltpu.async_copy(
            buf.at[b, pl.ds(0, MAIN_CHUNK)], tab.at[pl.ds(r0, MAIN_CHUNK)], ssems[b]
        )
    stores[0].wait()
    stores[1].wait()
    @pl.when(s < MAIN_NCH - 15 * NS)
    def _():
        r0 = pl.multiple_of((15 * NS + s) * MAIN_CHUNK, 8)
        pltpu.sync_copy(main_hbm.at[pl.ds(r0, MAIN_CHUNK)], buf.at[0, pl.ds(0, MAIN_CHUNK)])
        pltpu.sync_copy(buf.at[0, pl.ds(0, MAIN_CHUNK)], tab.at[pl.ds(r0, MAIN_CHUNK)])
    f0 = pl.multiple_of(s * FUS_PER_S, 8)
    pltpu.sync_copy(fus_hbm.at[pl.ds(f0, FUS_PER_S)], buf.at[1, pl.ds(0, FUS_PER_S)])
    pltpu.sync_copy(
        buf.at[1, pl.ds(0, FUS_PER_S)],
        tab.at[pl.ds(pl.multiple_of(V + s * FUS_PER_S, 8), FUS_PER_S)],
    )

    plsc.subcore_barrier()

    base = w * TOK_PER_W
    pltpu.sync_copy(
        idx_hbm.at[pl.ds(pl.multiple_of(w * IDX_ROWS_W, 8), IDX_ROWS_W)], idx_v
    )
    osems = (ssem0, ssem1)

    def pair(i, _):
        for b in range(2):
            u = 2 * i + b
            @pl.when(i >= 1)
            def _():
                pltpu.make_async_copy(
                    buf.at[b], out_hbm.at[pl.ds(0, UNIT)], osems[b]
                ).wait()

            copies = [
                pltpu.async_copy(
                    tab.at[idx_v.at[u * GPU_ + j]],
                    buf.at[b, pl.ds(j * IDX_W, IDX_W)],
                    gsem,
                )
                for j in range(GPU_)
            ]
            for cp in copies:
                cp.wait()
            t0 = pl.multiple_of(base + u * UNIT, 8)
            pltpu.async_copy(buf.at[b], out_hbm.at[pl.ds(t0, UNIT)], osems[b])
        return _

    lax.fori_loop(0, N_UNIT // 2, pair, None)
    for b in range(2):
        pltpu.make_async_copy(buf.at[b], out_hbm.at[pl.ds(0, UNIT)], osems[b]).wait()


def kernel(input, embedding_weight, fusion_weight):
    idx = input.reshape(NTOK // IDX_W, IDX_W).astype(jnp.int32)
    out, _table = _fused(embedding_weight, fusion_weight, idx)
    return out.reshape(B, S, D)
